# initial kernel scaffold (unmeasured)
import jax
import jax.numpy as jnp
from jax import lax
from jax.experimental import pallas as pl
from jax.experimental.pallas import tpu as pltpu


def kernel(
    x,
):
    def body(*refs):
        pass

    out_shape = jax.ShapeDtypeStruct(..., jnp.float32)
    return pl.pallas_call(body, out_shape=out_shape)(...)


# baseline (device time: 357743 ns/iter reference)
import jax
import jax.numpy as jnp
from jax import lax
from jax.experimental import pallas as pl
from jax.experimental.pallas import tpu as pltpu

N_DEV = 4
M, N = 16384, 1024
HALF = M // 2
QUART = HALF // 2
EIGHTH = HALF // 4


def kernel(x):
    xb = x.astype(jnp.bfloat16)

    def body(x_hbm, out_ref, rs1_buf, rs2_buf, load_sem, send_sems, recv_sems):
        r = lax.axis_index("i")
        nbr_a = r ^ 1
        nbr_b = 3 - r

        load = pltpu.make_async_copy(x_hbm, out_ref, load_sem)
        load.start()

        barrier = pltpu.get_barrier_semaphore()
        for nbr in (nbr_a, nbr_b):
            pl.semaphore_signal(
                barrier, inc=1,
                device_id=(nbr,), device_id_type=pl.DeviceIdType.MESH,
            )
        pl.semaphore_wait(barrier, 2)
        load.wait()

        halves = []
        for h in range(2):
            base = h * HALF
            if h == 0:
                k1 = (r ^ (r >> 1)) & 1
                k2 = (r >> 1) & 1
                partners = (nbr_a, nbr_b, nbr_b, nbr_a)
            else:
                k1 = (r >> 1) & 1
                k2 = r & 1
                partners = (nbr_b, nbr_a, nbr_a, nbr_b)
            off1_keep = base + k1 * QUART
            off1_send = base + (1 - k1) * QUART
            off2_keep = off1_keep + k2 * EIGHTH
            off2_send = off1_keep + (1 - k2) * EIGHTH
            halves.append(dict(
                partners=partners,
                off1_keep=off1_keep, off1_send=off1_send,
                off2_keep=off2_keep, off2_send=off2_send,
            ))

        def exchange(step, h, src_off, nrows, dst_slice, partner):
            rdma = pltpu.make_async_remote_copy(
                src_ref=out_ref.at[pl.ds(src_off, nrows)],
                dst_ref=dst_slice,
                send_sem=send_sems.at[h, step],
                recv_sem=recv_sems.at[h, step],
                device_id=(partner,),
                device_id_type=pl.DeviceIdType.MESH,
            )
            rdma.start()
            return rdma

        rdmas = [
            exchange(0, h, cfg["off1_send"], QUART, rs1_buf.at[h],
                     cfg["partners"][0])
            for h, cfg in enumerate(halves)
        ]
        for h, cfg in enumerate(halves):
            rdmas[h].wait_recv()
            off = cfg["off1_keep"]
            out_ref[pl.ds(off, QUART), :] = (
                out_ref[pl.ds(off, QUART), :] + rs1_buf[h]
            )
            rdmas[h].wait_send()

        rdmas = [
            exchange(1, h, cfg["off2_send"], EIGHTH, rs2_buf.at[h],
                     cfg["partners"][1])
            for h, cfg in enumerate(halves)
        ]
        for h, cfg in enumerate(halves):
            rdmas[h].wait_recv()
            off = cfg["off2_keep"]
            out_ref[pl.ds(off, EIGHTH), :] = (
                out_ref[pl.ds(off, EIGHTH), :] + rs2_buf[h]
            )
            rdmas[h].wait_send()

        rdmas = [
            exchange(2, h, cfg["off2_keep"], EIGHTH,
                     out_ref.at[pl.ds(cfg["off2_keep"], EIGHTH)],
                     cfg["partners"][2])
            for h, cfg in enumerate(halves)
        ]
        for rdma in rdmas:
            rdma.wait_recv()
            rdma.wait_send()

        rdmas = [
            exchange(3, h, cfg["off1_keep"], QUART,
                     out_ref.at[pl.ds(cfg["off1_keep"], QUART)],
                     cfg["partners"][3])
            for h, cfg in enumerate(halves)
        ]
        for rdma in rdmas:
            rdma.wait_recv()
            rdma.wait_send()

    return pl.pallas_call(
        body,
        out_shape=jax.ShapeDtypeStruct((M, N), jnp.bfloat16),
        in_specs=[pl.BlockSpec(memory_space=pl.ANY)],
        out_specs=pl.BlockSpec(memory_space=pltpu.VMEM),
        scratch_shapes=[
            pltpu.VMEM((2, QUART, N), jnp.bfloat16),
            pltpu.VMEM((2, EIGHTH, N), jnp.bfloat16),
            pltpu.SemaphoreType.DMA,
            pltpu.SemaphoreType.DMA((2, 4)),
            pltpu.SemaphoreType.DMA((2, 4)),
        ],
        compiler_params=pltpu.CompilerParams(
            collective_id=0,
            vmem_limit_bytes=60 * 1024 * 1024,
        ),
    )(xb)


# device time: 317503 ns/iter; 1.1267x vs baseline; 1.1267x over previous
import jax
import jax.numpy as jnp
from jax import lax
from jax.experimental import pallas as pl
from jax.experimental.pallas import tpu as pltpu

N_DEV = 4
M, N = 16384, 1024
HALF = M // 2
QUART = HALF // 2
EIGHTH = HALF // 4
CHUNK = 512


def kernel(x):
    def body(x_hbm, out_hbm, acc, rs1_buf, rs2_buf, stage,
             stage_sems, copy_sems, send_sems, recv_sems):
        r = lax.axis_index("i")
        nbr_a = r ^ 1
        nbr_b = 3 - r

        barrier = pltpu.get_barrier_semaphore()
        for nbr in (nbr_a, nbr_b):
            pl.semaphore_signal(
                barrier, inc=1,
                device_id=(nbr,), device_id_type=pl.DeviceIdType.MESH,
            )
        pl.semaphore_wait(barrier, 2)

        halves = []
        for h in range(2):
            base = h * HALF
            if h == 0:
                k1 = (r ^ (r >> 1)) & 1
                k2 = (r >> 1) & 1
                partners = (nbr_a, nbr_b, nbr_b, nbr_a)
            else:
                k1 = (r >> 1) & 1
                k2 = r & 1
                partners = (nbr_b, nbr_a, nbr_a, nbr_b)
            off1_keep = base + k1 * QUART
            off1_send = base + (1 - k1) * QUART
            off2_keep = off1_keep + k2 * EIGHTH
            off2_send = off1_keep + (1 - k2) * EIGHTH
            halves.append(dict(
                partners=partners, k2=k2,
                off1_keep=off1_keep, off1_send=off1_send,
                off2_keep=off2_keep, off2_send=off2_send,
            ))

        def convert_block(row_off):
            n = QUART // CHUNK
            cps = {}

            def start(c):
                cp = pltpu.make_async_copy(
                    x_hbm.at[pl.ds(row_off + c * CHUNK, CHUNK)],
                    stage.at[c % 2],
                    stage_sems.at[c % 2],
                )
                cp.start()
                cps[c] = cp

            start(0)
            for c in range(n):
                if c + 1 < n:
                    start(c + 1)
                cps[c].wait()
                acc[pl.ds(row_off + c * CHUNK, CHUNK), :] = (
                    stage[c % 2].astype(jnp.bfloat16)
                )

        def exchange(step, h, src_slice, dst_slice, partner):
            rdma = pltpu.make_async_remote_copy(
                src_ref=src_slice,
                dst_ref=dst_slice,
                send_sem=send_sems.at[h, step],
                recv_sem=recv_sems.at[h, step],
                device_id=(partner,),
                device_id_type=pl.DeviceIdType.MESH,
            )
            rdma.start()
            return rdma

        all_rdmas = []

        rs1 = [None, None]
        for h, cfg in enumerate(halves):
            convert_block(cfg["off1_send"])
            rs1[h] = exchange(
                0, h,
                acc.at[pl.ds(cfg["off1_send"], QUART)],
                rs1_buf.at[h],
                cfg["partners"][0],
            )
        for cfg in halves:
            convert_block(cfg["off1_keep"])
        all_rdmas += rs1

        rs2 = [None, None]
        for h, cfg in enumerate(halves):
            rs1[h].wait_recv()
            k2 = cfg["k2"]
            acc[pl.ds(cfg["off2_send"], EIGHTH), :] = (
                acc[pl.ds(cfg["off2_send"], EIGHTH), :]
                + rs1_buf[h, pl.ds((1 - k2) * EIGHTH, EIGHTH), :]
            )
            rs2[h] = exchange(
                1, h,
                acc.at[pl.ds(cfg["off2_send"], EIGHTH)],
                rs2_buf.at[h],
                cfg["partners"][1],
            )
            acc[pl.ds(cfg["off2_keep"], EIGHTH), :] = (
                acc[pl.ds(cfg["off2_keep"], EIGHTH), :]
                + rs1_buf[h, pl.ds(k2 * EIGHTH, EIGHTH), :]
            )
        all_rdmas += rs2

        ag1 = [None, None]
        own_cp = [None, None]
        for h, cfg in enumerate(halves):
            rs2[h].wait_recv()
            acc[pl.ds(cfg["off2_keep"], EIGHTH), :] = (
                acc[pl.ds(cfg["off2_keep"], EIGHTH), :] + rs2_buf[h]
            )
            ag1[h] = exchange(
                2, h,
                acc.at[pl.ds(cfg["off2_keep"], EIGHTH)],
                out_hbm.at[pl.ds(cfg["off2_keep"], EIGHTH)],
                cfg["partners"][2],
            )
            own_cp[h] = pltpu.make_async_copy(
                acc.at[pl.ds(cfg["off2_keep"], EIGHTH)],
                out_hbm.at[pl.ds(cfg["off2_keep"], EIGHTH)],
                copy_sems.at[h],
            )
            own_cp[h].start()
        all_rdmas += ag1

        ag2 = [None, None]
        for h, cfg in enumerate(halves):
            ag1[h].wait_recv()
            own_cp[h].wait()
            ag2[h] = exchange(
                3, h,
                out_hbm.at[pl.ds(cfg["off1_keep"], QUART)],
                out_hbm.at[pl.ds(cfg["off1_keep"], QUART)],
                cfg["partners"][3],
            )
        all_rdmas += ag2

        for rdma in ag2:
            rdma.wait_recv()
        for rdma in all_rdmas:
            rdma.wait_send()

    return pl.pallas_call(
        body,
        out_shape=jax.ShapeDtypeStruct((M, N), jnp.bfloat16),
        in_specs=[pl.BlockSpec(memory_space=pl.ANY)],
        out_specs=pl.BlockSpec(memory_space=pl.ANY),
        scratch_shapes=[
            pltpu.VMEM((M, N), jnp.bfloat16),
            pltpu.VMEM((2, QUART, N), jnp.bfloat16),
            pltpu.VMEM((2, EIGHTH, N), jnp.bfloat16),
            pltpu.VMEM((2, CHUNK, N), jnp.float32),
            pltpu.SemaphoreType.DMA((2,)),
            pltpu.SemaphoreType.DMA((2,)),
            pltpu.SemaphoreType.DMA((2, 4)),
            pltpu.SemaphoreType.DMA((2, 4)),
        ],
        compiler_params=pltpu.CompilerParams(
            collective_id=0,
            vmem_limit_bytes=63 * 1024 * 1024,
        ),
    )(x)


# device time: 301505 ns/iter; 1.1865x vs baseline; 1.0531x over previous
import jax
import jax.numpy as jnp
from jax import lax
from jax.experimental import pallas as pl
from jax.experimental.pallas import tpu as pltpu

N_DEV = 4
M, N = 16384, 1024
HALF = M // 2
QUART = HALF // 2
EIGHTH = HALF // 4
CHUNK = 512
NPIECE = QUART // CHUNK

RS2, AG1, AG2A, AG2B = 0, 1, 2, 3


def kernel(x):
    def body(x_hbm, out_hbm, acc, rs1_buf, rs2_buf, stage,
             stage_sems, copy_sems, rs1_send, rs1_recv, m_send, m_recv):
        r = lax.axis_index("i")
        nbr_a = r ^ 1
        nbr_b = 3 - r

        barrier = pltpu.get_barrier_semaphore()
        for nbr in (nbr_a, nbr_b):
            pl.semaphore_signal(
                barrier, inc=1,
                device_id=(nbr,), device_id_type=pl.DeviceIdType.MESH,
            )
        pl.semaphore_wait(barrier, 2)

        halves = []
        for h in range(2):
            base = h * HALF
            if h == 0:
                k1 = (r ^ (r >> 1)) & 1
                k2 = (r >> 1) & 1
                partners = (nbr_a, nbr_b, nbr_b, nbr_a)
            else:
                k1 = (r >> 1) & 1
                k2 = r & 1
                partners = (nbr_b, nbr_a, nbr_a, nbr_b)
            off1_keep = base + k1 * QUART
            off1_send = base + (1 - k1) * QUART
            off2_keep = off1_keep + k2 * EIGHTH
            off2_send = off1_keep + (1 - k2) * EIGHTH
            first_rel = (1 - k2) * EIGHTH if h == 0 else k2 * EIGHTH
            second_rel = k2 * EIGHTH if h == 0 else (1 - k2) * EIGHTH
            halves.append(dict(
                partners=partners, k2=k2,
                off1_keep=off1_keep, off1_send=off1_send,
                off2_keep=off2_keep, off2_send=off2_send,
                first_rel=first_rel, second_rel=second_rel,
            ))

        def remote(src_slice, dst_slice, s_sem, r_sem, partner):
            rdma = pltpu.make_async_remote_copy(
                src_ref=src_slice, dst_ref=dst_slice,
                send_sem=s_sem, recv_sem=r_sem,
                device_id=(partner,),
                device_id_type=pl.DeviceIdType.MESH,
            )
            rdma.start()
            return rdma

        entries = []
        for p in range(NPIECE):
            for h, cfg in enumerate(halves):
                rel = (cfg["first_rel"] + p * CHUNK if p < NPIECE // 2
                       else cfg["second_rel"] + (p - NPIECE // 2) * CHUNK)
                entries.append((cfg["off1_send"] + rel, (h, p, rel)))
        for cfg in halves:
            for c in range(NPIECE):
                entries.append((cfg["off1_keep"] + c * CHUNK, None))

        rs1_rdmas = [[None] * NPIECE, [None] * NPIECE]
        cps = {}

        def start_chunk(i):
            cp = pltpu.make_async_copy(
                x_hbm.at[pl.ds(entries[i][0], CHUNK)],
                stage.at[i % 2],
                stage_sems.at[i % 2],
            )
            cp.start()
            cps[i] = cp

        start_chunk(0)
        for i, (row, send) in enumerate(entries):
            if i + 1 < len(entries):
                start_chunk(i + 1)
            cps[i].wait()
            acc[pl.ds(row, CHUNK), :] = stage[i % 2].astype(jnp.bfloat16)
            if send is not None:
                h, p, rel = send
                rs1_rdmas[h][p] = remote(
                    acc.at[pl.ds(row, CHUNK)],
                    rs1_buf.at[h, pl.ds(rel, CHUNK)],
                    rs1_send.at[h, p], rs1_recv.at[h, p],
                    halves[h]["partners"][0],
                )

        rs2 = [None, None]
        for h, cfg in enumerate(halves):
            k2 = cfg["k2"]
            for p in range(NPIECE // 2):
                rs1_rdmas[h][p].wait_recv()
            acc[pl.ds(cfg["off2_send"], EIGHTH), :] = (
                acc[pl.ds(cfg["off2_send"], EIGHTH), :]
                + rs1_buf[h, pl.ds((1 - k2) * EIGHTH, EIGHTH), :]
            )
            rs2[h] = remote(
                acc.at[pl.ds(cfg["off2_send"], EIGHTH)],
                rs2_buf.at[h],
                m_send.at[h, RS2], m_recv.at[h, RS2],
                cfg["partners"][1],
            )
        for h, cfg in enumerate(halves):
            k2 = cfg["k2"]
            for p in range(NPIECE // 2, NPIECE):
                rs1_rdmas[h][p].wait_recv()
            acc[pl.ds(cfg["off2_keep"], EIGHTH), :] = (
                acc[pl.ds(cfg["off2_keep"], EIGHTH), :]
                + rs1_buf[h, pl.ds(k2 * EIGHTH, EIGHTH), :]
            )

        ag1 = [None, None]
        ag2a = [None, None]
        own_cp = [None, None]
        for h, cfg in enumerate(halves):
            rs2[h].wait_recv()
            acc[pl.ds(cfg["off2_keep"], EIGHTH), :] = (
                acc[pl.ds(cfg["off2_keep"], EIGHTH), :] + rs2_buf[h]
            )
            ag1[h] = remote(
                acc.at[pl.ds(cfg["off2_keep"], EIGHTH)],
                out_hbm.at[pl.ds(cfg["off2_keep"], EIGHTH)],
                m_send.at[h, AG1], m_recv.at[h, AG1],
                cfg["partners"][2],
            )
            ag2a[h] = remote(
                acc.at[pl.ds(cfg["off2_keep"], EIGHTH)],
                out_hbm.at[pl.ds(cfg["off2_keep"], EIGHTH)],
                m_send.at[h, AG2A], m_recv.at[h, AG2A],
                cfg["partners"][3],
            )
            own_cp[h] = pltpu.make_async_copy(
                acc.at[pl.ds(cfg["off2_keep"], EIGHTH)],
                out_hbm.at[pl.ds(cfg["off2_keep"], EIGHTH)],
                copy_sems.at[h],
            )
            own_cp[h].start()

        ag2b = [None, None]
        for h, cfg in enumerate(halves):
            ag1[h].wait_recv()
            ag2b[h] = remote(
                out_hbm.at[pl.ds(cfg["off2_send"], EIGHTH)],
                out_hbm.at[pl.ds(cfg["off2_send"], EIGHTH)],
                m_send.at[h, AG2B], m_recv.at[h, AG2B],
                cfg["partners"][3],
            )

        for h in range(2):
            ag2a[h].wait_recv()
            ag2b[h].wait_recv()
            own_cp[h].wait()
        for h in range(2):
            for p in range(NPIECE):
                rs1_rdmas[h][p].wait_send()
            for rdma in (rs2[h], ag1[h], ag2a[h], ag2b[h]):
                rdma.wait_send()

    return pl.pallas_call(
        body,
        out_shape=jax.ShapeDtypeStruct((M, N), jnp.bfloat16),
        in_specs=[pl.BlockSpec(memory_space=pl.ANY)],
        out_specs=pl.BlockSpec(memory_space=pl.ANY),
        scratch_shapes=[
            pltpu.VMEM((M, N), jnp.bfloat16),
            pltpu.VMEM((2, QUART, N), jnp.bfloat16),
            pltpu.VMEM((2, EIGHTH, N), jnp.bfloat16),
            pltpu.VMEM((2, CHUNK, N), jnp.float32),
            pltpu.SemaphoreType.DMA((2,)),
            pltpu.SemaphoreType.DMA((2,)),
            pltpu.SemaphoreType.DMA((2, NPIECE)),
            pltpu.SemaphoreType.DMA((2, NPIECE)),
            pltpu.SemaphoreType.DMA((2, 4)),
            pltpu.SemaphoreType.DMA((2, 4)),
        ],
        compiler_params=pltpu.CompilerParams(
            collective_id=0,
            vmem_limit_bytes=63 * 1024 * 1024,
        ),
    )(x)


# device time: 298790 ns/iter; 1.1973x vs baseline; 1.0091x over previous
import jax
import jax.numpy as jnp
from jax import lax
from jax.experimental import pallas as pl
from jax.experimental.pallas import tpu as pltpu

N_DEV = 4
M, N = 16384, 1024
HALF = M // 2
QUART = HALF // 2
EIGHTH = HALF // 4
CHUNK = 512
NPIECE = QUART // CHUNK

PIECE2 = 1024

RS2_0, RS2_1, AG1_0, AG1_1, AG2A, AG2B_0, AG2B_1 = range(7)


def kernel(x):
    def body(x_hbm, out_hbm, acc, rs1_buf, rs2_buf, stage,
             stage_sems, copy_sems, rs1_send, rs1_recv, m_send, m_recv):
        r = lax.axis_index("i")
        nbr_a = r ^ 1
        nbr_b = 3 - r

        barrier = pltpu.get_barrier_semaphore()
        for nbr in (nbr_a, nbr_b):
            pl.semaphore_signal(
                barrier, inc=1,
                device_id=(nbr,), device_id_type=pl.DeviceIdType.MESH,
            )
        pl.semaphore_wait(barrier, 2)

        halves = []
        for h in range(2):
            base = h * HALF
            if h == 0:
                k1 = (r ^ (r >> 1)) & 1
                k2 = (r >> 1) & 1
                partners = (nbr_a, nbr_b, nbr_b, nbr_a)
            else:
                k1 = (r >> 1) & 1
                k2 = r & 1
                partners = (nbr_b, nbr_a, nbr_a, nbr_b)
            off1_keep = base + k1 * QUART
            off1_send = base + (1 - k1) * QUART
            off2_keep = off1_keep + k2 * EIGHTH
            off2_send = off1_keep + (1 - k2) * EIGHTH
            first_rel = (1 - k2) * EIGHTH if h == 0 else k2 * EIGHTH
            second_rel = k2 * EIGHTH if h == 0 else (1 - k2) * EIGHTH
            halves.append(dict(
                partners=partners, k2=k2,
                off1_keep=off1_keep, off1_send=off1_send,
                off2_keep=off2_keep, off2_send=off2_send,
                first_rel=first_rel, second_rel=second_rel,
            ))

        def remote(src_slice, dst_slice, s_sem, r_sem, partner):
            rdma = pltpu.make_async_remote_copy(
                src_ref=src_slice, dst_ref=dst_slice,
                send_sem=s_sem, recv_sem=r_sem,
                device_id=(partner,),
                device_id_type=pl.DeviceIdType.MESH,
            )
            rdma.start()
            return rdma

        entries = []
        for p in range(NPIECE):
            for h, cfg in enumerate(halves):
                rel = (cfg["first_rel"] + p * CHUNK if p < NPIECE // 2
                       else cfg["second_rel"] + (p - NPIECE // 2) * CHUNK)
                entries.append((cfg["off1_send"] + rel, (h, p, rel)))
        for cfg in halves:
            for c in range(NPIECE):
                entries.append((cfg["off1_keep"] + c * CHUNK, None))

        rs1_rdmas = [[None] * NPIECE, [None] * NPIECE]
        cps = {}

        def start_chunk(i):
            cp = pltpu.make_async_copy(
                x_hbm.at[pl.ds(entries[i][0], CHUNK)],
                stage.at[i % 2],
                stage_sems.at[i % 2],
            )
            cp.start()
            cps[i] = cp

        start_chunk(0)
        for i, (row, send) in enumerate(entries):
            if i + 1 < len(entries):
                start_chunk(i + 1)
            cps[i].wait()
            acc[pl.ds(row, CHUNK), :] = stage[i % 2].astype(jnp.bfloat16)
            if send is not None:
                h, p, rel = send
                rs1_rdmas[h][p] = remote(
                    acc.at[pl.ds(row, CHUNK)],
                    rs1_buf.at[h, pl.ds(rel, CHUNK)],
                    rs1_send.at[h, p], rs1_recv.at[h, p],
                    halves[h]["partners"][0],
                )

        rs2p = [[None, None], [None, None]]
        for p in range(NPIECE // 2):
            for h, cfg in enumerate(halves):
                rs1_rdmas[h][p].wait_recv()
                row = cfg["off2_send"] + p * CHUNK
                acc[pl.ds(row, CHUNK), :] = (
                    acc[pl.ds(row, CHUNK), :]
                    + rs1_buf[h, pl.ds((1 - cfg["k2"]) * EIGHTH + p * CHUNK,
                                       CHUNK), :]
                )
            if p % 2 == 1:
                j = p // 2
                for h, cfg in enumerate(halves):
                    rs2p[h][j] = remote(
                        acc.at[pl.ds(cfg["off2_send"] + j * PIECE2, PIECE2)],
                        rs2_buf.at[h, pl.ds(j * PIECE2, PIECE2)],
                        m_send.at[h, RS2_0 + j], m_recv.at[h, RS2_0 + j],
                        cfg["partners"][1],
                    )
        for p in range(NPIECE // 2, NPIECE):
            q = p - NPIECE // 2
            for h, cfg in enumerate(halves):
                rs1_rdmas[h][p].wait_recv()
                row = cfg["off2_keep"] + q * CHUNK
                acc[pl.ds(row, CHUNK), :] = (
                    acc[pl.ds(row, CHUNK), :]
                    + rs1_buf[h, pl.ds(cfg["k2"] * EIGHTH + q * CHUNK,
                                       CHUNK), :]
                )

        ag1p = [[None, None], [None, None]]
        ag2a = [None, None]
        own_cp = [None, None]
        for j in range(2):
            for h, cfg in enumerate(halves):
                rs2p[h][j].wait_recv()
                row = cfg["off2_keep"] + j * PIECE2
                acc[pl.ds(row, PIECE2), :] = (
                    acc[pl.ds(row, PIECE2), :]
                    + rs2_buf[h, pl.ds(j * PIECE2, PIECE2), :]
                )
                ag1p[h][j] = remote(
                    acc.at[pl.ds(row, PIECE2)],
                    out_hbm.at[pl.ds(row, PIECE2)],
                    m_send.at[h, AG1_0 + j], m_recv.at[h, AG1_0 + j],
                    cfg["partners"][2],
                )
                if j == 1:
                    ag2a[h] = remote(
                        acc.at[pl.ds(cfg["off2_keep"], EIGHTH)],
                        out_hbm.at[pl.ds(cfg["off2_keep"], EIGHTH)],
                        m_send.at[h, AG2A], m_recv.at[h, AG2A],
                        cfg["partners"][3],
                    )
                    own_cp[h] = pltpu.make_async_copy(
                        acc.at[pl.ds(cfg["off2_keep"], EIGHTH)],
                        out_hbm.at[pl.ds(cfg["off2_keep"], EIGHTH)],
                        copy_sems.at[h],
                    )
                    own_cp[h].start()

        ag2bp = [[None, None], [None, None]]
        for j in range(2):
            for h, cfg in enumerate(halves):
                ag1p[h][j].wait_recv()
                row = cfg["off2_send"] + j * PIECE2
                ag2bp[h][j] = remote(
                    out_hbm.at[pl.ds(row, PIECE2)],
                    out_hbm.at[pl.ds(row, PIECE2)],
                    m_send.at[h, AG2B_0 + j], m_recv.at[h, AG2B_0 + j],
                    cfg["partners"][3],
                )

        for h in range(2):
            ag2a[h].wait_recv()
            for j in range(2):
                ag2bp[h][j].wait_recv()
            own_cp[h].wait()
        for h in range(2):
            for p in range(NPIECE):
                rs1_rdmas[h][p].wait_send()
            for j in range(2):
                for rdma in (rs2p[h][j], ag1p[h][j], ag2bp[h][j]):
                    rdma.wait_send()
            ag2a[h].wait_send()

    return pl.pallas_call(
        body,
        out_shape=jax.ShapeDtypeStruct((M, N), jnp.bfloat16),
        in_specs=[pl.BlockSpec(memory_space=pl.ANY)],
        out_specs=pl.BlockSpec(memory_space=pl.ANY),
        scratch_shapes=[
            pltpu.VMEM((M, N), jnp.bfloat16),
            pltpu.VMEM((2, QUART, N), jnp.bfloat16),
            pltpu.VMEM((2, EIGHTH, N), jnp.bfloat16),
            pltpu.VMEM((2, CHUNK, N), jnp.float32),
            pltpu.SemaphoreType.DMA((2,)),
            pltpu.SemaphoreType.DMA((2,)),
            pltpu.SemaphoreType.DMA((2, NPIECE)),
            pltpu.SemaphoreType.DMA((2, NPIECE)),
            pltpu.SemaphoreType.DMA((2, 7)),
            pltpu.SemaphoreType.DMA((2, 7)),
        ],
        compiler_params=pltpu.CompilerParams(
            collective_id=0,
            vmem_limit_bytes=63 * 1024 * 1024,
        ),
    )(x)
